# baseline (speedup 1.0000x reference)
import functools

import jax
import jax.numpy as jnp
from jax import lax
from jax.experimental import pallas as pl
from jax.experimental.pallas import tpu as pltpu
from jax.experimental.pallas import tpu_sc as plsc

_NUM_CLASSES = 5
_H = _W = 13
_S = _H * _W
_SP = 176
_A = 3
_ANCHOR_W = (0.05, 0.1, 0.15)
_CH = 2 * _NUM_CLASSES
_N = 10
_NSUB = 16
_LN2 = 0.6931471805599453


def _logf(x):
    bits = lax.bitcast_convert_type(x, jnp.int32)
    e = ((bits >> 23) & 0xFF) - 127
    f = lax.bitcast_convert_type((bits & 0x7FFFFF) | 0x3F800000, jnp.float32)
    big = f > 1.41421356
    f = jnp.where(big, f * 0.5, f)
    e = (e + big.astype(jnp.int32)).astype(jnp.float32)
    z = f - 1.0
    y = z * z
    p = jnp.full_like(z, 7.0376836292e-2)
    for c in (-1.1514610310e-1, 1.1676998740e-1, -1.2420140846e-1,
              1.4249322787e-1, -1.6668057665e-1, 2.0000714765e-1,
              -2.4999993993e-1, 3.3333331174e-1):
        p = p * z + c
    return z + z * y * p - 0.5 * y + e * _LN2


def _bce0(x):
    return jnp.maximum(x, 0.0) + _logf(1.0 + jnp.exp(-jnp.abs(x)))


def _splat(vec, j):
    return jnp.full((16,), vec[j], vec.dtype)


def _partials_body(pred_hbm, cx_hbm, cy_hbm, w_hbm, h_hbm, lab_hbm, out_hbm,
                   pred_v, cx_v, cy_v, w_v, h_v, lab_v, part_v):
    cid = lax.axis_index("c")
    sid = lax.axis_index("s")

    @pl.when(cid == 0)
    def _core0():
        lane = lax.iota(jnp.int32, 16)

        def sample_loss(b):
            pltpu.sync_copy(pred_hbm.at[b], pred_v)
            pltpu.sync_copy(cx_hbm.at[b], cx_v)
            pltpu.sync_copy(cy_hbm.at[b], cy_v)
            pltpu.sync_copy(w_hbm.at[b], w_v)
            pltpu.sync_copy(h_hbm.at[b], h_v)
            pltpu.sync_copy(lab_hbm.at[b], lab_v)
            cx = cx_v[...]
            cy = cy_v[...]
            w = w_v[...]
            h = h_v[...]
            lab = lab_v[...]

            valid = (cx > 0) & (cx < 1) & (cy > 0) & (cy < 1) & (w > 0) & (h > 0)
            cxs = cx * _W
            cys = cy * _H
            gx = cxs.astype(jnp.int32)
            gx = gx - (gx.astype(jnp.float32) > cxs).astype(jnp.int32)
            gy = cys.astype(jnp.int32)
            gy = gy - (gy.astype(jnp.float32) > cys).astype(jnp.int32)
            gx = jnp.minimum(jnp.maximum(gx, 0), _W - 1)
            gy = jnp.minimum(jnp.maximum(gy, 0), _H - 1)
            ious = []
            for aw in _ANCHOR_W:
                inter = jnp.minimum(w, aw) * jnp.minimum(h, aw)
                ious.append(inter / (w * h + aw * aw - inter))
            best = jnp.where(ious[1] > ious[0], 1, 0).astype(jnp.int32)
            best = jnp.where(ious[2] > jnp.maximum(ious[0], ious[1]), 2, best)
            awb = jnp.where(best == 0, _ANCHOR_W[0],
                            jnp.where(best == 1, _ANCHOR_W[1], _ANCHOR_W[2]))
            tx = cxs - gx.astype(jnp.float32)
            ty = cys - gy.astype(jnp.float32)
            tw = _logf(w / awb + 1e-16)
            th = _logf(h / awb + 1e-16)
            scell = gy * _W + gx
            boxkey = best * _S + scell
            cls_ok = valid & (lab >= 0) & (lab < _NUM_CLASSES)
            pairkey = boxkey * _NUM_CLASSES + jnp.minimum(
                jnp.maximum(lab, 0), _NUM_CLASSES - 1)

            vval = valid.astype(jnp.int32)
            cval = cls_ok.astype(jnp.int32)
            killed = jnp.zeros((16,), jnp.bool_)
            pkilled = jnp.zeros((16,), jnp.bool_)
            for j in range(1, _N):
                later = lane < j
                killed = killed | ((_splat(vval, j) > 0)
                                   & (boxkey == _splat(boxkey, j)) & later)
                pkilled = pkilled | ((_splat(cval, j) > 0)
                                     & (pairkey == _splat(pairkey, j)) & later)
            win = valid & (~killed)
            winpair = cls_ok & (~pkilled)
            winf = win.astype(jnp.float32)
            cntv = jnp.full((16,), jnp.sum(winf))
            pwv = (float(_A * _S) - cntv) / (cntv + 1e-16)

            dense = jnp.zeros((16,), jnp.float32)
            for a in range(_A):
                def chunk(t, acc):
                    return acc + _bce0(pred_v[a * _CH, pl.ds(t * 16, 16)])
                dense = lax.fori_loop(0, _SP // 16, chunk, dense)
            padv = jnp.where(lane == 0, (_A * (_SP - _S)) * float(_LN2), 0.0)

            chbase = best * _CH
            o = plsc.load_gather(pred_v, [chbase, scell])
            corrv = jnp.where(win, pwv * (_bce0(o) - o) - _bce0(o), 0.0)

            bb = jnp.zeros((16,), jnp.float32)
            for k, val in enumerate((tx, ty, tw, th)):
                pb = plsc.load_gather(pred_v, [chbase + (1 + k), scell])
                d = pb - val
                bb = bb + jnp.where(win, d * d, 0.0)
            clsbce = jnp.zeros((16,), jnp.float32)
            for k in range(_NUM_CLASSES):
                pc = plsc.load_gather(pred_v, [chbase + (_NUM_CLASSES + k), scell])
                clsbce = clsbce + jnp.where(win, _bce0(pc), 0.0)
            pcl = plsc.load_gather(
                pred_v, [chbase + _NUM_CLASSES + jnp.minimum(
                    jnp.maximum(lab, 0), _NUM_CLASSES - 1), scell])
            pairv = jnp.where(winpair, pcl, 0.0)

            hasv = cntv > 0
            w_bb = jnp.where(hasv, 5.0 / (4.0 * cntv + 1e-30), 0.0)
            w_cls = jnp.where(hasv, 2.0 / (float(_NUM_CLASSES) * cntv + 1e-30),
                              0.0)
            return ((dense - padv + corrv) * (2.0 / float(_A * _S))
                    + bb * w_bb + (clsbce - pairv) * w_cls)

        def body(k, acc):
            return acc + sample_loss(sid * 4 + k)

        accv = lax.fori_loop(0, 4, body, jnp.zeros((16,), jnp.float32))
        part_v[...] = accv * (1.0 / 64.0)
        pltpu.sync_copy(part_v, out_hbm.at[sid])


def _reduce_body(parts_hbm, out_hbm, parts_v, res_v):
    cid = lax.axis_index("c")
    sid = lax.axis_index("s")

    @pl.when((cid == 0) & (sid == 0))
    def _t0():
        lane = lax.iota(jnp.int32, 16)
        pltpu.sync_copy(parts_hbm, parts_v)
        acc = parts_v[0]
        for r in range(1, _NSUB):
            acc = acc + parts_v[r]
        res_v[...] = jnp.where(lane == 0, jnp.sum(acc), 0.0)
        pltpu.sync_copy(res_v, out_hbm)


def kernel(predictions, targets_boxes, targets_labels):
    B = predictions.shape[0]
    pred = jnp.pad(predictions.reshape(B, _A * _CH, _S),
                   ((0, 0), (0, 0), (0, _SP - _S)))
    pad10 = ((0, 0), (0, 16 - _N))
    cx = jnp.pad(targets_boxes[:, :, 0], pad10)
    cy = jnp.pad(targets_boxes[:, :, 1], pad10)
    w = jnp.pad(targets_boxes[:, :, 2], pad10)
    h = jnp.pad(targets_boxes[:, :, 3], pad10)
    lab = jnp.pad(targets_labels, pad10)

    mesh = plsc.VectorSubcoreMesh(core_axis_name="c", subcore_axis_name="s")
    parts = functools.partial(
        pl.kernel,
        mesh=mesh,
        compiler_params=pltpu.CompilerParams(needs_layout_passes=False),
        out_type=jax.ShapeDtypeStruct((_NSUB, 16), jnp.float32),
        scratch_types=[
            pltpu.VMEM((_A * _CH, _SP), jnp.float32),
            pltpu.VMEM((16,), jnp.float32),
            pltpu.VMEM((16,), jnp.float32),
            pltpu.VMEM((16,), jnp.float32),
            pltpu.VMEM((16,), jnp.float32),
            pltpu.VMEM((16,), jnp.int32),
            pltpu.VMEM((16,), jnp.float32),
        ],
    )(_partials_body)(pred, cx, cy, w, h, lab)

    out = functools.partial(
        pl.kernel,
        mesh=mesh,
        compiler_params=pltpu.CompilerParams(needs_layout_passes=False),
        out_type=jax.ShapeDtypeStruct((16,), jnp.float32),
        scratch_types=[
            pltpu.VMEM((_NSUB, 16), jnp.float32),
            pltpu.VMEM((16,), jnp.float32),
        ],
    )(_reduce_body)(parts)
    return out[0]
